# baseline (device time: 87023 ns/iter reference)
import jax
import jax.numpy as jnp
from jax import lax
from jax.experimental import pallas as pl
from jax.experimental.pallas import tpu as pltpu

N_DEV = 4
NSUB = 4

_R_SLOT = (2, 1, 3)
_L_SLOT = (2, 0, 3)


def kernel(x, w_mat):
    m_glob, k_loc = x.shape
    n = w_mat.shape[1]
    m_per = m_glob // N_DEV
    n_half = n // 2
    n_sub = n_half // NSUB

    def body(x_hbm, w_ref, out_ref, wb_ref, xb_ref, fbuf, rcomm, lcomm,
             rsend_sems, rrecv_sems, lsend_sems, lrecv_sems, xsems):
        my = lax.axis_index("i")
        left = lax.rem(my + N_DEV - 1, N_DEV)
        right = lax.rem(my + 1, N_DEV)

        def xcopy(c, slot, sem_i):
            return pltpu.make_async_copy(
                x_hbm.at[pl.ds(c * m_per, m_per), :],
                fbuf.at[slot],
                xsems.at[sem_i],
            )

        cp = [
            xcopy(lax.rem(my + N_DEV - 1, N_DEV), 0, 0),
            xcopy(lax.rem(my + 1, N_DEV), 1, 1),
            xcopy(lax.rem(my + 2, N_DEV), 0, 2),
            xcopy(my, 1, 3),
        ]
        cp[0].start()
        cp[1].start()

        barrier_sem = pltpu.get_barrier_semaphore()
        for nbr in [left, right]:
            pl.semaphore_signal(
                barrier_sem, inc=1,
                device_id=(nbr,), device_id_type=pl.DeviceIdType.MESH,
            )
        pl.semaphore_wait(barrier_sem, 2)

        def make(comm, send_sems, recv_sems, tgt, h, j):
            return pltpu.make_async_remote_copy(
                src_ref=comm.at[h % 2, j],
                dst_ref=comm.at[(h + 1) % 2, j],
                send_sem=send_sems.at[h % 2, j],
                recv_sem=recv_sems.at[(h + 1) % 2, j],
                device_id=(tgt,),
                device_id_type=pl.DeviceIdType.MESH,
            )

        r_rdma = [[make(rcomm, rsend_sems, rrecv_sems, right, h, j)
                   for j in range(NSUB)] for h in range(N_DEV - 1)]
        l_rdma = [[make(lcomm, lsend_sems, lrecv_sems, left, h, j)
                   for j in range(NSUB)] for h in range(N_DEV - 1)]

        def rcols(j):
            return slice(j * n_sub, (j + 1) * n_sub)

        def lcols(j):
            return slice(n_half + j * n_sub, n_half + (j + 1) * n_sub)

        cp[0].wait()
        xb_ref[0] = fbuf[0].astype(jnp.bfloat16)
        cp[2].start()
        for j in range(NSUB):
            wb_ref[:, rcols(j)] = w_ref[:, rcols(j)].astype(jnp.bfloat16)
            rcomm[0, j, :, :] = jnp.dot(
                xb_ref[0], wb_ref[:, rcols(j)],
                preferred_element_type=jnp.float32,
            ).astype(jnp.bfloat16)
            r_rdma[0][j].start()

        cp[1].wait()
        xb_ref[1] = fbuf[1].astype(jnp.bfloat16)
        cp[3].start()
        for j in range(NSUB):
            wb_ref[:, lcols(j)] = w_ref[:, lcols(j)].astype(jnp.bfloat16)
            lcomm[0, j, :, :] = jnp.dot(
                xb_ref[1], wb_ref[:, lcols(j)],
                preferred_element_type=jnp.float32,
            ).astype(jnp.bfloat16)
            l_rdma[0][j].start()

        for h in range(N_DEV - 1):
            rs = (h + 1) % 2
            last = h == N_DEV - 2
            if h == 0:
                cp[2].wait()
                xb_ref[2] = fbuf[0].astype(jnp.bfloat16)
            elif h == 1:
                cp[3].wait()
                xb_ref[3] = fbuf[1].astype(jnp.bfloat16)
            xc_r = xb_ref[_R_SLOT[h]]
            xc_l = xb_ref[_L_SLOT[h]]
            for j in range(NSUB):
                p_rj = jnp.dot(xc_r, wb_ref[:, rcols(j)],
                               preferred_element_type=jnp.float32)
                if h >= 1:
                    r_rdma[h - 1][j].wait_send()
                r_rdma[h][j].wait_recv()
                acc = rcomm[rs, j, :, :].astype(jnp.float32) + p_rj
                if not last:
                    rcomm[rs, j, :, :] = acc.astype(jnp.bfloat16)
                    r_rdma[h + 1][j].start()
                else:
                    out_ref[:, rcols(j)] = acc * jax.nn.sigmoid(acc)

                p_lj = jnp.dot(xc_l, wb_ref[:, lcols(j)],
                               preferred_element_type=jnp.float32)
                if h >= 1:
                    l_rdma[h - 1][j].wait_send()
                l_rdma[h][j].wait_recv()
                acc = lcomm[rs, j, :, :].astype(jnp.float32) + p_lj
                if not last:
                    lcomm[rs, j, :, :] = acc.astype(jnp.bfloat16)
                    l_rdma[h + 1][j].start()
                else:
                    out_ref[:, lcols(j)] = acc * jax.nn.sigmoid(acc)

        for j in range(NSUB):
            r_rdma[N_DEV - 2][j].wait_send()
            l_rdma[N_DEV - 2][j].wait_send()

    return pl.pallas_call(
        body,
        out_shape=jax.ShapeDtypeStruct((m_per, n), jnp.float32),
        in_specs=[
            pl.BlockSpec(memory_space=pltpu.MemorySpace.HBM),
            pl.BlockSpec(memory_space=pltpu.VMEM),
        ],
        out_specs=pl.BlockSpec(memory_space=pltpu.VMEM),
        scratch_shapes=[
            pltpu.VMEM((k_loc, n), jnp.bfloat16),
            pltpu.VMEM((N_DEV, m_per, k_loc), jnp.bfloat16),
            pltpu.VMEM((2, m_per, k_loc), jnp.float32),
            pltpu.VMEM((2, NSUB, m_per, n_sub), jnp.bfloat16),
            pltpu.VMEM((2, NSUB, m_per, n_sub), jnp.bfloat16),
            pltpu.SemaphoreType.DMA((2, NSUB)),
            pltpu.SemaphoreType.DMA((2, NSUB)),
            pltpu.SemaphoreType.DMA((2, NSUB)),
            pltpu.SemaphoreType.DMA((2, NSUB)),
            pltpu.SemaphoreType.DMA((4,)),
        ],
        compiler_params=pltpu.CompilerParams(
            collective_id=0,
            vmem_limit_bytes=44 * 1024 * 1024,
        ),
    )(x, w_mat)


# device time: 84788 ns/iter; 1.0264x vs baseline; 1.0264x over previous
import jax
import jax.numpy as jnp
from jax import lax
from jax.experimental import pallas as pl
from jax.experimental.pallas import tpu as pltpu

N_DEV = 4
NSUB = 4

_R_SLOT = (2, 1, 3)
_L_SLOT = (2, 0, 3)


def kernel(x, w_mat):
    m_glob, k_loc = x.shape
    n = w_mat.shape[1]
    m_per = m_glob // N_DEV
    n_half = n // 2
    n_sub = n_half // NSUB

    def body(x_hbm, w_ref, out_ref, wb_ref, xb_ref, fbuf, rcomm, lcomm,
             rsend_sems, rrecv_sems, lsend_sems, lrecv_sems, xsems):
        my = lax.axis_index("i")
        left = lax.rem(my + N_DEV - 1, N_DEV)
        right = lax.rem(my + 1, N_DEV)

        def xcopy(c, slot, sem_i):
            return pltpu.make_async_copy(
                x_hbm.at[pl.ds(c * m_per, m_per), :],
                fbuf.at[slot],
                xsems.at[sem_i],
            )

        cp = [
            xcopy(lax.rem(my + N_DEV - 1, N_DEV), 0, 0),
            xcopy(lax.rem(my + 1, N_DEV), 1, 1),
            xcopy(lax.rem(my + 2, N_DEV), 0, 2),
            xcopy(my, 1, 3),
        ]
        cp[0].start()
        cp[1].start()

        barrier_sem = pltpu.get_barrier_semaphore()
        for nbr in [left, right]:
            pl.semaphore_signal(
                barrier_sem, inc=1,
                device_id=(nbr,), device_id_type=pl.DeviceIdType.MESH,
            )
        pl.semaphore_wait(barrier_sem, 2)

        def make(comm, send_sems, recv_sems, tgt, h, j):
            return pltpu.make_async_remote_copy(
                src_ref=comm.at[h % 2, j],
                dst_ref=comm.at[(h + 1) % 2, j],
                send_sem=send_sems.at[h % 2, j],
                recv_sem=recv_sems.at[(h + 1) % 2, j],
                device_id=(tgt,),
                device_id_type=pl.DeviceIdType.MESH,
            )

        r_rdma = [[make(rcomm, rsend_sems, rrecv_sems, right, h, j)
                   for j in range(NSUB)] for h in range(N_DEV - 1)]
        l_rdma = [[make(lcomm, lsend_sems, lrecv_sems, left, h, j)
                   for j in range(NSUB)] for h in range(N_DEV - 1)]

        def rcols(j):
            return slice(j * n_sub, (j + 1) * n_sub)

        def lcols(j):
            return slice(n_half + j * n_sub, n_half + (j + 1) * n_sub)

        cp[0].wait()
        xb_ref[0] = fbuf[0].astype(jnp.bfloat16)
        cp[2].start()
        cp[1].wait()
        xb_ref[1] = fbuf[1].astype(jnp.bfloat16)
        cp[3].start()
        for j in range(NSUB):
            wb_ref[:, rcols(j)] = w_ref[:, rcols(j)].astype(jnp.bfloat16)
            rcomm[0, j, :, :] = jnp.dot(
                xb_ref[0], wb_ref[:, rcols(j)],
                preferred_element_type=jnp.float32,
            ).astype(jnp.bfloat16)
            r_rdma[0][j].start()
            wb_ref[:, lcols(j)] = w_ref[:, lcols(j)].astype(jnp.bfloat16)
            lcomm[0, j, :, :] = jnp.dot(
                xb_ref[1], wb_ref[:, lcols(j)],
                preferred_element_type=jnp.float32,
            ).astype(jnp.bfloat16)
            l_rdma[0][j].start()

        for h in range(N_DEV - 1):
            rs = (h + 1) % 2
            last = h == N_DEV - 2
            if h == 0:
                cp[2].wait()
                xb_ref[2] = fbuf[0].astype(jnp.bfloat16)
            elif h == 1:
                cp[3].wait()
                xb_ref[3] = fbuf[1].astype(jnp.bfloat16)
            xc_r = xb_ref[_R_SLOT[h]]
            xc_l = xb_ref[_L_SLOT[h]]
            for j in range(NSUB):
                p_rj = jnp.dot(xc_r, wb_ref[:, rcols(j)],
                               preferred_element_type=jnp.float32)
                if h >= 1:
                    r_rdma[h - 1][j].wait_send()
                r_rdma[h][j].wait_recv()
                acc = rcomm[rs, j, :, :].astype(jnp.float32) + p_rj
                if not last:
                    rcomm[rs, j, :, :] = acc.astype(jnp.bfloat16)
                    r_rdma[h + 1][j].start()
                else:
                    out_ref[:, rcols(j)] = acc * jax.nn.sigmoid(acc)

                p_lj = jnp.dot(xc_l, wb_ref[:, lcols(j)],
                               preferred_element_type=jnp.float32)
                if h >= 1:
                    l_rdma[h - 1][j].wait_send()
                l_rdma[h][j].wait_recv()
                acc = lcomm[rs, j, :, :].astype(jnp.float32) + p_lj
                if not last:
                    lcomm[rs, j, :, :] = acc.astype(jnp.bfloat16)
                    l_rdma[h + 1][j].start()
                else:
                    out_ref[:, lcols(j)] = acc * jax.nn.sigmoid(acc)

        for j in range(NSUB):
            r_rdma[N_DEV - 2][j].wait_send()
            l_rdma[N_DEV - 2][j].wait_send()

    return pl.pallas_call(
        body,
        out_shape=jax.ShapeDtypeStruct((m_per, n), jnp.float32),
        in_specs=[
            pl.BlockSpec(memory_space=pltpu.MemorySpace.HBM),
            pl.BlockSpec(memory_space=pltpu.VMEM),
        ],
        out_specs=pl.BlockSpec(memory_space=pltpu.VMEM),
        scratch_shapes=[
            pltpu.VMEM((k_loc, n), jnp.bfloat16),
            pltpu.VMEM((N_DEV, m_per, k_loc), jnp.bfloat16),
            pltpu.VMEM((2, m_per, k_loc), jnp.float32),
            pltpu.VMEM((2, NSUB, m_per, n_sub), jnp.bfloat16),
            pltpu.VMEM((2, NSUB, m_per, n_sub), jnp.bfloat16),
            pltpu.SemaphoreType.DMA((2, NSUB)),
            pltpu.SemaphoreType.DMA((2, NSUB)),
            pltpu.SemaphoreType.DMA((2, NSUB)),
            pltpu.SemaphoreType.DMA((2, NSUB)),
            pltpu.SemaphoreType.DMA((4,)),
        ],
        compiler_params=pltpu.CompilerParams(
            collective_id=0,
            vmem_limit_bytes=44 * 1024 * 1024,
        ),
    )(x, w_mat)


# device time: 84684 ns/iter; 1.0276x vs baseline; 1.0012x over previous
import jax
import jax.numpy as jnp
from jax import lax
from jax.experimental import pallas as pl
from jax.experimental.pallas import tpu as pltpu

N_DEV = 4
NSUB = 8

_R_SLOT = (2, 1, 3)
_L_SLOT = (2, 0, 3)


def kernel(x, w_mat):
    m_glob, k_loc = x.shape
    n = w_mat.shape[1]
    m_per = m_glob // N_DEV
    n_half = n // 2
    n_sub = n_half // NSUB

    def body(x_hbm, w_ref, out_ref, wb_ref, xb_ref, fbuf, rcomm, lcomm,
             rsend_sems, rrecv_sems, lsend_sems, lrecv_sems, xsems):
        my = lax.axis_index("i")
        left = lax.rem(my + N_DEV - 1, N_DEV)
        right = lax.rem(my + 1, N_DEV)

        def xcopy(c, slot, sem_i):
            return pltpu.make_async_copy(
                x_hbm.at[pl.ds(c * m_per, m_per), :],
                fbuf.at[slot],
                xsems.at[sem_i],
            )

        cp = [
            xcopy(lax.rem(my + N_DEV - 1, N_DEV), 0, 0),
            xcopy(lax.rem(my + 1, N_DEV), 1, 1),
            xcopy(lax.rem(my + 2, N_DEV), 0, 2),
            xcopy(my, 1, 3),
        ]
        cp[0].start()
        cp[1].start()

        barrier_sem = pltpu.get_barrier_semaphore()
        for nbr in [left, right]:
            pl.semaphore_signal(
                barrier_sem, inc=1,
                device_id=(nbr,), device_id_type=pl.DeviceIdType.MESH,
            )
        pl.semaphore_wait(barrier_sem, 2)

        def make(comm, send_sems, recv_sems, tgt, h, j):
            return pltpu.make_async_remote_copy(
                src_ref=comm.at[h % 2, j],
                dst_ref=comm.at[(h + 1) % 2, j],
                send_sem=send_sems.at[h % 2, j],
                recv_sem=recv_sems.at[(h + 1) % 2, j],
                device_id=(tgt,),
                device_id_type=pl.DeviceIdType.MESH,
            )

        r_rdma = [[make(rcomm, rsend_sems, rrecv_sems, right, h, j)
                   for j in range(NSUB)] for h in range(N_DEV - 1)]
        l_rdma = [[make(lcomm, lsend_sems, lrecv_sems, left, h, j)
                   for j in range(NSUB)] for h in range(N_DEV - 1)]

        def rcols(j):
            return slice(j * n_sub, (j + 1) * n_sub)

        def lcols(j):
            return slice(n_half + j * n_sub, n_half + (j + 1) * n_sub)

        cp[0].wait()
        xb_ref[0] = fbuf[0].astype(jnp.bfloat16)
        cp[2].start()
        cp[1].wait()
        xb_ref[1] = fbuf[1].astype(jnp.bfloat16)
        cp[3].start()
        for j in range(NSUB):
            wb_ref[:, rcols(j)] = w_ref[:, rcols(j)].astype(jnp.bfloat16)
            rcomm[0, j, :, :] = jnp.dot(
                xb_ref[0], wb_ref[:, rcols(j)],
                preferred_element_type=jnp.float32,
            ).astype(jnp.bfloat16)
            r_rdma[0][j].start()
            wb_ref[:, lcols(j)] = w_ref[:, lcols(j)].astype(jnp.bfloat16)
            lcomm[0, j, :, :] = jnp.dot(
                xb_ref[1], wb_ref[:, lcols(j)],
                preferred_element_type=jnp.float32,
            ).astype(jnp.bfloat16)
            l_rdma[0][j].start()

        for h in range(N_DEV - 1):
            rs = (h + 1) % 2
            last = h == N_DEV - 2
            if h == 0:
                cp[2].wait()
                xb_ref[2] = fbuf[0].astype(jnp.bfloat16)
            elif h == 1:
                cp[3].wait()
                xb_ref[3] = fbuf[1].astype(jnp.bfloat16)
            xc_r = xb_ref[_R_SLOT[h]]
            xc_l = xb_ref[_L_SLOT[h]]
            for j in range(NSUB):
                p_rj = jnp.dot(xc_r, wb_ref[:, rcols(j)],
                               preferred_element_type=jnp.float32)
                if h >= 1:
                    r_rdma[h - 1][j].wait_send()
                r_rdma[h][j].wait_recv()
                acc = rcomm[rs, j, :, :].astype(jnp.float32) + p_rj
                if not last:
                    rcomm[rs, j, :, :] = acc.astype(jnp.bfloat16)
                    r_rdma[h + 1][j].start()
                else:
                    out_ref[:, rcols(j)] = acc * jax.nn.sigmoid(acc)

                p_lj = jnp.dot(xc_l, wb_ref[:, lcols(j)],
                               preferred_element_type=jnp.float32)
                if h >= 1:
                    l_rdma[h - 1][j].wait_send()
                l_rdma[h][j].wait_recv()
                acc = lcomm[rs, j, :, :].astype(jnp.float32) + p_lj
                if not last:
                    lcomm[rs, j, :, :] = acc.astype(jnp.bfloat16)
                    l_rdma[h + 1][j].start()
                else:
                    out_ref[:, lcols(j)] = acc * jax.nn.sigmoid(acc)

        for j in range(NSUB):
            r_rdma[N_DEV - 2][j].wait_send()
            l_rdma[N_DEV - 2][j].wait_send()

    return pl.pallas_call(
        body,
        out_shape=jax.ShapeDtypeStruct((m_per, n), jnp.float32),
        in_specs=[
            pl.BlockSpec(memory_space=pltpu.MemorySpace.HBM),
            pl.BlockSpec(memory_space=pltpu.VMEM),
        ],
        out_specs=pl.BlockSpec(memory_space=pltpu.VMEM),
        scratch_shapes=[
            pltpu.VMEM((k_loc, n), jnp.bfloat16),
            pltpu.VMEM((N_DEV, m_per, k_loc), jnp.bfloat16),
            pltpu.VMEM((2, m_per, k_loc), jnp.float32),
            pltpu.VMEM((2, NSUB, m_per, n_sub), jnp.bfloat16),
            pltpu.VMEM((2, NSUB, m_per, n_sub), jnp.bfloat16),
            pltpu.SemaphoreType.DMA((2, NSUB)),
            pltpu.SemaphoreType.DMA((2, NSUB)),
            pltpu.SemaphoreType.DMA((2, NSUB)),
            pltpu.SemaphoreType.DMA((2, NSUB)),
            pltpu.SemaphoreType.DMA((4,)),
        ],
        compiler_params=pltpu.CompilerParams(
            collective_id=0,
            vmem_limit_bytes=44 * 1024 * 1024,
        ),
    )(x, w_mat)


# device time: 83457 ns/iter; 1.0427x vs baseline; 1.0147x over previous
import jax
import jax.numpy as jnp
from jax import lax
from jax.experimental import pallas as pl
from jax.experimental.pallas import tpu as pltpu

N_DEV = 4
NSUB = 8

_R_SLOT = (2, 1, 3)
_L_SLOT = (2, 0, 3)


def kernel(x, w_mat):
    m_glob, k_loc = x.shape
    n = w_mat.shape[1]
    m_per = m_glob // N_DEV
    n_half = n // 2
    n_sub = n_half // NSUB

    def body(x_hbm, w_ref, out_ref, wb_ref, xb_ref, fbuf, rcomm, lcomm,
             rsend_sems, rrecv_sems, lsend_sems, lrecv_sems, xsems):
        my = lax.axis_index("i")
        left = lax.rem(my + N_DEV - 1, N_DEV)
        right = lax.rem(my + 1, N_DEV)

        def xcopy(c, slot, sem_i):
            return pltpu.make_async_copy(
                x_hbm.at[pl.ds(c * m_per, m_per), :],
                fbuf.at[slot],
                xsems.at[sem_i],
            )

        cp = [
            xcopy(lax.rem(my + N_DEV - 1, N_DEV), 0, 0),
            xcopy(lax.rem(my + 1, N_DEV), 1, 1),
            xcopy(lax.rem(my + 2, N_DEV), 0, 2),
            xcopy(my, 1, 3),
        ]
        cp[0].start()
        cp[1].start()

        barrier_sem = pltpu.get_barrier_semaphore()
        for nbr in [left, right]:
            pl.semaphore_signal(
                barrier_sem, inc=1,
                device_id=(nbr,), device_id_type=pl.DeviceIdType.MESH,
            )
        pl.semaphore_wait(barrier_sem, 2)

        def make(comm, send_sems, recv_sems, tgt, h, j):
            return pltpu.make_async_remote_copy(
                src_ref=comm.at[h % 2, j],
                dst_ref=comm.at[(h + 1) % 2, j],
                send_sem=send_sems.at[h % 2, j],
                recv_sem=recv_sems.at[(h + 1) % 2, j],
                device_id=(tgt,),
                device_id_type=pl.DeviceIdType.MESH,
            )

        r_rdma = [[make(rcomm, rsend_sems, rrecv_sems, right, h, j)
                   for j in range(NSUB)] for h in range(N_DEV - 1)]
        l_rdma = [[make(lcomm, lsend_sems, lrecv_sems, left, h, j)
                   for j in range(NSUB)] for h in range(N_DEV - 1)]

        def rcols(j):
            return slice(j * n_sub, (j + 1) * n_sub)

        def lcols(j):
            return slice(n_half + j * n_sub, n_half + (j + 1) * n_sub)

        def r_init(j):
            wb_ref[:, rcols(j)] = w_ref[:, rcols(j)].astype(jnp.bfloat16)
            rcomm[0, j, :, :] = jnp.dot(
                xb_ref[0], wb_ref[:, rcols(j)],
                preferred_element_type=jnp.float32,
            ).astype(jnp.bfloat16)
            r_rdma[0][j].start()

        def l_init(j):
            wb_ref[:, lcols(j)] = w_ref[:, lcols(j)].astype(jnp.bfloat16)
            lcomm[0, j, :, :] = jnp.dot(
                xb_ref[1], wb_ref[:, lcols(j)],
                preferred_element_type=jnp.float32,
            ).astype(jnp.bfloat16)
            l_rdma[0][j].start()

        cp[0].wait()
        xb_ref[0] = fbuf[0].astype(jnp.bfloat16)
        cp[2].start()
        r_init(0)
        cp[1].wait()
        xb_ref[1] = fbuf[1].astype(jnp.bfloat16)
        cp[3].start()
        l_init(0)
        for j in range(1, NSUB):
            r_init(j)
            l_init(j)

        for h in range(N_DEV - 1):
            rs = (h + 1) % 2
            last = h == N_DEV - 2
            if h == 0:
                cp[2].wait()
                xb_ref[2] = fbuf[0].astype(jnp.bfloat16)
            elif h == 1:
                cp[3].wait()
                xb_ref[3] = fbuf[1].astype(jnp.bfloat16)
            xc_r = xb_ref[_R_SLOT[h]]
            xc_l = xb_ref[_L_SLOT[h]]
            for j in range(NSUB):
                p_rj = jnp.dot(xc_r, wb_ref[:, rcols(j)],
                               preferred_element_type=jnp.float32)
                if h >= 1:
                    r_rdma[h - 1][j].wait_send()
                r_rdma[h][j].wait_recv()
                acc = rcomm[rs, j, :, :].astype(jnp.float32) + p_rj
                if not last:
                    rcomm[rs, j, :, :] = acc.astype(jnp.bfloat16)
                    r_rdma[h + 1][j].start()
                else:
                    out_ref[:, rcols(j)] = acc * jax.nn.sigmoid(acc)

                p_lj = jnp.dot(xc_l, wb_ref[:, lcols(j)],
                               preferred_element_type=jnp.float32)
                if h >= 1:
                    l_rdma[h - 1][j].wait_send()
                l_rdma[h][j].wait_recv()
                acc = lcomm[rs, j, :, :].astype(jnp.float32) + p_lj
                if not last:
                    lcomm[rs, j, :, :] = acc.astype(jnp.bfloat16)
                    l_rdma[h + 1][j].start()
                else:
                    out_ref[:, lcols(j)] = acc * jax.nn.sigmoid(acc)

        for j in range(NSUB):
            r_rdma[N_DEV - 2][j].wait_send()
            l_rdma[N_DEV - 2][j].wait_send()

    return pl.pallas_call(
        body,
        out_shape=jax.ShapeDtypeStruct((m_per, n), jnp.float32),
        in_specs=[
            pl.BlockSpec(memory_space=pltpu.MemorySpace.HBM),
            pl.BlockSpec(memory_space=pltpu.VMEM),
        ],
        out_specs=pl.BlockSpec(memory_space=pltpu.VMEM),
        scratch_shapes=[
            pltpu.VMEM((k_loc, n), jnp.bfloat16),
            pltpu.VMEM((N_DEV, m_per, k_loc), jnp.bfloat16),
            pltpu.VMEM((2, m_per, k_loc), jnp.float32),
            pltpu.VMEM((2, NSUB, m_per, n_sub), jnp.bfloat16),
            pltpu.VMEM((2, NSUB, m_per, n_sub), jnp.bfloat16),
            pltpu.SemaphoreType.DMA((2, NSUB)),
            pltpu.SemaphoreType.DMA((2, NSUB)),
            pltpu.SemaphoreType.DMA((2, NSUB)),
            pltpu.SemaphoreType.DMA((2, NSUB)),
            pltpu.SemaphoreType.DMA((4,)),
        ],
        compiler_params=pltpu.CompilerParams(
            collective_id=0,
            vmem_limit_bytes=44 * 1024 * 1024,
        ),
    )(x, w_mat)
